# baseline (speedup 1.0000x reference)
import functools

import jax
import jax.numpy as jnp
import numpy as np
from jax.experimental import pallas as pl
from jax.experimental.pallas import tpu as pltpu

NN = 10000
NE = 160000
NIN = 5
MSG = 128
NGRAPH = 64
HS = 64
BNC = float(1.0 / np.sqrt(1.0 + 1e-3))
_TRANS = np.array([0.0, 0.0, -200.0, 10000.0, 0.0], dtype=np.float32)
_SCALE = np.array([100.0, 100.0, 100.0, 2500.0, 0.25], dtype=np.float32)


def _relu(v):
    return jnp.maximum(v, 0.0)


def _k0_body(x_ref, w_ref, b_ref, o_ref):
    o_ref[...] = (
        jnp.dot(x_ref[...], w_ref[...], preferred_element_type=jnp.float32)
        + b_ref[...]
    )


def _init_tables(x, w0c, b0c):
    n = x.shape[0]
    bn = 1000
    return pl.pallas_call(
        _k0_body,
        grid=(n // bn,),
        in_specs=[
            pl.BlockSpec((bn, NIN), lambda i: (i, 0)),
            pl.BlockSpec((NIN, 512), lambda i: (0, 0)),
            pl.BlockSpec((1, 512), lambda i: (0, 0)),
        ],
        out_specs=pl.BlockSpec((bn, 512), lambda i: (i, 0)),
        out_shape=jax.ShapeDtypeStruct((n, 512), jnp.float32),
    )(x, w0c, b0c)


def _kb_body(p_ref, q_ref, w_ref, b_ref, o_ref):
    m1 = _relu(p_ref[...] + q_ref[...])
    o_ref[...] = _relu(
        jnp.dot(m1, w_ref[...], preferred_element_type=jnp.float32) + b_ref[...]
    )


def _edge_mlp(pg, qg, w2, b2):
    e = pg.shape[0]
    be = 1600
    return pl.pallas_call(
        _kb_body,
        grid=(e // be,),
        in_specs=[
            pl.BlockSpec((be, MSG), lambda i: (i, 0)),
            pl.BlockSpec((be, MSG), lambda i: (i, 0)),
            pl.BlockSpec((MSG, MSG), lambda i: (0, 0)),
            pl.BlockSpec((1, MSG), lambda i: (0, 0)),
        ],
        out_specs=pl.BlockSpec((be, MSG), lambda i: (i, 0)),
        out_shape=jax.ShapeDtypeStruct((e, MSG), jnp.float32),
    )(pg, qg, w2, b2)


def _kd_body(make_tables, mn_ref, mx_ref, sm_ref, sq_ref, rc_ref, wu1_ref,
             bu1_ref, wu2_ref, bu2_ref, *rest):
    if make_tables:
        (a_ref, bw_ref, pb_ref, xc_ref, h_ref, p_ref, q_ref) = rest
    else:
        (h_ref,) = rest
    rc = rc_ref[...]
    mean = sm_ref[...] * rc
    var = sq_ref[...] * rc - mean * mean
    emb = jnp.concatenate([mn_ref[...], mx_ref[...], mean, var], axis=1)
    u = _relu(jnp.dot(emb, wu1_ref[...], preferred_element_type=jnp.float32)
              + bu1_ref[...])
    u = _relu(jnp.dot(u, wu2_ref[...], preferred_element_type=jnp.float32)
              + bu2_ref[...])
    h = u * BNC
    h_ref[...] = h
    if make_tables:
        xc = xc_ref[...]
        p_ref[...] = (
            jnp.dot(h, a_ref[...], preferred_element_type=jnp.float32)
            + pb_ref[...] - xc
        )
        q_ref[...] = (
            jnp.dot(h, bw_ref[...], preferred_element_type=jnp.float32) + xc
        )


def _node_update(mn, mx, sm, sq, rcnt, wu1, bu1, wu2, bu2,
                 nxt=None):
    n = mn.shape[0]
    bn = 1000
    row = lambda i: (i, 0)
    full = lambda i: (0, 0)
    in_specs = [
        pl.BlockSpec((bn, MSG), row),
        pl.BlockSpec((bn, MSG), row),
        pl.BlockSpec((bn, MSG), row),
        pl.BlockSpec((bn, MSG), row),
        pl.BlockSpec((bn, 1), row),
        pl.BlockSpec((512, MSG), full),
        pl.BlockSpec((1, MSG), full),
        pl.BlockSpec((MSG, MSG), full),
        pl.BlockSpec((1, MSG), full),
    ]
    args = [mn, mx, sm, sq, rcnt, wu1, bu1, wu2, bu2]
    make_tables = nxt is not None
    if make_tables:
        a, bw, pb, xc = nxt
        in_specs += [
            pl.BlockSpec((MSG, MSG), full),
            pl.BlockSpec((MSG, MSG), full),
            pl.BlockSpec((1, MSG), full),
            pl.BlockSpec((bn, MSG), row),
        ]
        args += [a, bw, pb, xc]
        out_specs = [pl.BlockSpec((bn, MSG), row)] * 3
        out_shape = [jax.ShapeDtypeStruct((n, MSG), jnp.float32)] * 3
    else:
        out_specs = pl.BlockSpec((bn, MSG), row)
        out_shape = jax.ShapeDtypeStruct((n, MSG), jnp.float32)
    return pl.pallas_call(
        functools.partial(_kd_body, make_tables),
        grid=(n // bn,),
        in_specs=in_specs,
        out_specs=out_specs,
        out_shape=out_shape,
    )(*args)


def _kf_body(mx_ref, mn_ref, sm_ref, rg_ref, d1_ref, db1_ref, d2_ref, db2_ref,
             d3_ref, db3_ref, hw_ref, hb_ref, o_ref):
    psum = sm_ref[...]
    g = jnp.concatenate([mx_ref[...], psum * rg_ref[...], psum, mn_ref[...]],
                        axis=1)
    t = jnp.dot(g, d1_ref[...], preferred_element_type=jnp.float32) + db1_ref[...]
    t = jnp.dot(t, d2_ref[...], preferred_element_type=jnp.float32) + db2_ref[...]
    t = jnp.dot(t, d3_ref[...], preferred_element_type=jnp.float32) + db3_ref[...]
    cols = []
    for k in range(4):
        w0 = hw_ref[k, :192, :64]
        w1 = hw_ref[k, 192:256, :64]
        w2 = hw_ref[k, 256:320, 0:1]
        b0 = hb_ref[k, 0:1, :64]
        b1 = hb_ref[k, 1:2, :64]
        b2 = hb_ref[k, 2:3, 0:1]
        u = (jnp.dot(t, w0, preferred_element_type=jnp.float32) + b0) * BNC
        u = (jnp.dot(u, w1, preferred_element_type=jnp.float32) + b1) * BNC
        cols.append(jnp.dot(u, w2, preferred_element_type=jnp.float32) + b2)
    o = jnp.concatenate(cols, axis=1)
    d3 = o[:, 0:3]
    nrm = jnp.sqrt(jnp.sum(d3 * d3, axis=1, keepdims=True))
    pos = nrm > 0.0
    dirv = jnp.where(pos, d3 / jnp.where(pos, nrm, 1.0), 0.0)
    o_ref[...] = jnp.concatenate([dirv, o[:, 3:4]], axis=1)


def _decode(gmx, gmn, gsm, rgcnt, d1, db1, d2, db2, d3, db3, hw, hb):
    full = lambda: (0, 0)
    return pl.pallas_call(
        _kf_body,
        out_shape=jax.ShapeDtypeStruct((NGRAPH, 4), jnp.float32),
    )(gmx, gmn, gsm, rgcnt, d1, db1, d2, db2, d3, db3, hw, hb)


def _prep(params):
    inv_s = 1.0 / _SCALE
    tcorr = -(_TRANS * inv_s)

    mp = params["mp"]
    A0 = mp[0]["Wm1"][0:NIN]
    B0 = mp[0]["Wm1"][NIN:2 * NIN]
    C0 = mp[0]["Wm1"][2 * NIN:3 * NIN]
    PW0 = A0 - BNC * C0
    QW0 = B0 + BNC * C0
    CW1 = BNC * mp[1]["Wm1"][2 * MSG:2 * MSG + NIN]
    CW2 = BNC * mp[2]["Wm1"][2 * MSG:2 * MSG + NIN]
    w0c = jnp.concatenate([PW0, QW0, CW1, CW2], axis=1) * inv_s[:, None]
    b0c = jnp.concatenate([
        mp[0]["bm1"] + tcorr @ PW0,
        tcorr @ QW0,
        tcorr @ CW1,
        tcorr @ CW2,
    ])[None, :]

    layers = []
    for l in range(3):
        lp = mp[l]
        d = {
            "Wm2": lp["Wm2"], "bm2": lp["bm2"][None, :],
            "Wu1": lp["Wu1"], "bu1": lp["bu1"][None, :],
            "Wu2": lp["Wu2"], "bu2": lp["bu2"][None, :],
        }
        if l < 2:
            nxt = mp[l + 1]
            d["A"] = nxt["Wm1"][0:MSG]
            d["Bw"] = nxt["Wm1"][MSG:2 * MSG]
            d["pb"] = nxt["bm1"][None, :]
        layers.append(d)

    dec = params["dec"]
    d1, db1 = dec[0][0] * BNC, dec[0][1][None, :] * BNC
    d2, db2 = dec[1][0] * BNC, dec[1][1][None, :] * BNC
    d3, db3 = dec[2][0] * BNC, dec[2][1][None, :] * BNC

    hws, hbs = [], []
    for sp in params["split"]:
        (W0, b0), (W1, b1), (W2, b2) = sp
        w = jnp.zeros((320, 64), jnp.float32)
        w = w.at[0:192, :].set(W0)
        w = w.at[192:256, :].set(W1)
        w = w.at[256:320, 0:1].set(W2)
        b = jnp.zeros((3, 64), jnp.float32)
        b = b.at[0, :].set(b0)
        b = b.at[1, :].set(b1)
        b = b.at[2, 0].set(b2[0])
        hws.append(w)
        hbs.append(b)
    hw = jnp.stack(hws)
    hb = jnp.stack(hbs)

    return dict(w0c=w0c, b0c=b0c, layers=layers,
                d1=d1, db1=db1, d2=d2, db2=db2, d3=d3, db3=db3,
                hw=hw, hb=hb)


def kernel(x, edge_index, graph_ids, params):
    pp = _prep(params)
    idx_i = edge_index[:, 0]
    idx_j = edge_index[:, 1]

    t0 = _init_tables(x, pp["w0c"], pp["b0c"])
    p = t0[:, 0:128]
    q = t0[:, 128:256]
    xc1 = t0[:, 256:384]
    xc2 = t0[:, 384:512]
    xcs = [None, xc1, xc2]

    cnt = jax.ops.segment_sum(jnp.ones((NE,), jnp.float32), idx_i,
                              num_segments=NN)
    rcnt = (1.0 / cnt)[:, None]

    h = None
    for l in range(3):
        lw = pp["layers"][l]
        pg = jnp.take(p, idx_i, axis=0)
        qg = jnp.take(q, idx_j, axis=0)
        m = _edge_mlp(pg, qg, lw["Wm2"], lw["bm2"])
        mn = jax.ops.segment_min(m, idx_i, num_segments=NN)
        mx = jax.ops.segment_max(m, idx_i, num_segments=NN)
        sm = jax.ops.segment_sum(m, idx_i, num_segments=NN)
        sq = jax.ops.segment_sum(m * m, idx_i, num_segments=NN)
        if l < 2:
            nxt = (lw["A"], lw["Bw"], lw["pb"], xcs[l + 1])
            h, p, q = _node_update(mn, mx, sm, sq, rcnt, lw["Wu1"], lw["bu1"],
                                   lw["Wu2"], lw["bu2"], nxt)
        else:
            h = _node_update(mn, mx, sm, sq, rcnt, lw["Wu1"], lw["bu1"],
                             lw["Wu2"], lw["bu2"])

    gmx = jax.ops.segment_max(h, graph_ids, num_segments=NGRAPH)
    gmn = jax.ops.segment_min(h, graph_ids, num_segments=NGRAPH)
    gsm = jax.ops.segment_sum(h, graph_ids, num_segments=NGRAPH)
    gcnt = jax.ops.segment_sum(jnp.ones((NN,), jnp.float32), graph_ids,
                               num_segments=NGRAPH)
    rgcnt = (1.0 / gcnt)[:, None]
    return _decode(gmx, gmn, gsm, rgcnt, pp["d1"], pp["db1"], pp["d2"],
                   pp["db2"], pp["d3"], pp["db3"], pp["hw"], pp["hb"])


# speedup vs baseline: 1.1310x; 1.1310x over previous
import functools

import jax
import jax.numpy as jnp
import numpy as np
from jax.experimental import pallas as pl
from jax.experimental.pallas import tpu as pltpu
from jax.experimental.pallas import tpu_sc as plsc

NN = 10000
NE = 160000
NP = 10240
EP = 163840
NIN = 5
MSG = 128
NGRAPH = 64
HS = 64
BNC = float(1.0 / np.sqrt(1.0 + 1e-3))
_TRANS = np.array([0.0, 0.0, -200.0, 10000.0, 0.0], dtype=np.float32)
_SCALE = np.array([100.0, 100.0, 100.0, 2500.0, 0.25], dtype=np.float32)


def _relu(v):
    return jnp.maximum(v, 0.0)


def _k0_body(x_ref, w_ref, b_ref, o_ref):
    o_ref[...] = (
        jnp.dot(x_ref[...], w_ref[...], preferred_element_type=jnp.float32)
        + b_ref[...]
    )


def _init_tables(x, w0c, b0c):
    n = x.shape[0]
    bn = 1024
    return pl.pallas_call(
        _k0_body,
        grid=(n // bn,),
        in_specs=[
            pl.BlockSpec((bn, NIN), lambda i: (i, 0)),
            pl.BlockSpec((NIN, 512), lambda i: (0, 0)),
            pl.BlockSpec((1, 512), lambda i: (0, 0)),
        ],
        out_specs=pl.BlockSpec((bn, 512), lambda i: (i, 0)),
        out_shape=jax.ShapeDtypeStruct((n, 512), jnp.float32),
    )(x, w0c, b0c)


GW = 128


def _sc_gather(p, q, ii, jj):
    mesh = plsc.VectorSubcoreMesh(core_axis_name="c", subcore_axis_name="s")
    ii2 = ii.reshape(1, EP)
    jj2 = jj.reshape(1, EP)

    @functools.partial(
        pl.kernel,
        out_type=[jax.ShapeDtypeStruct((EP, MSG), jnp.float32),
                  jax.ShapeDtypeStruct((EP, MSG), jnp.float32)],
        mesh=mesh,
    )
    def k(p_hbm, q_hbm, ii_hbm, jj_hbm, pg_hbm, qg_hbm):
        def body(ii_v, jj_v, po_v, qo_v):
            pltpu.sync_copy(p_hbm.at[ii_v.at[0]], po_v)
            pltpu.sync_copy(q_hbm.at[jj_v.at[0]], qo_v)

        pltpu.emit_pipeline(
            body,
            grid=(EP // GW,),
            in_specs=[pl.BlockSpec((1, GW), lambda i: (0, i)),
                      pl.BlockSpec((1, GW), lambda i: (0, i))],
            out_specs=[pl.BlockSpec((GW, MSG), lambda i: (i, 0)),
                       pl.BlockSpec((GW, MSG), lambda i: (i, 0))],
            core_axis_name=("c", "s"),
            dimension_semantics=(pltpu.PARALLEL,),
        )(ii_hbm, jj_hbm, pg_hbm, qg_hbm)

    return k(p, q, ii2, jj2)


def _kb_body(p_ref, q_ref, w_ref, b_ref, o_ref):
    m1 = _relu(p_ref[...] + q_ref[...])
    o_ref[...] = _relu(
        jnp.dot(m1, w_ref[...], preferred_element_type=jnp.float32) + b_ref[...]
    )


def _edge_mlp(pg, qg, w2, b2):
    e = pg.shape[0]
    be = 2048
    return pl.pallas_call(
        _kb_body,
        grid=(e // be,),
        in_specs=[
            pl.BlockSpec((be, MSG), lambda i: (i, 0)),
            pl.BlockSpec((be, MSG), lambda i: (i, 0)),
            pl.BlockSpec((MSG, MSG), lambda i: (0, 0)),
            pl.BlockSpec((1, MSG), lambda i: (0, 0)),
        ],
        out_specs=pl.BlockSpec((be, MSG), lambda i: (i, 0)),
        out_shape=jax.ShapeDtypeStruct((e, MSG), jnp.float32),
    )(pg, qg, w2, b2)


def _kd_body(make_tables, mn_ref, mx_ref, sm_ref, sq_ref, rc_ref, wu1_ref,
             bu1_ref, wu2_ref, bu2_ref, *rest):
    if make_tables:
        (a_ref, bw_ref, pb_ref, xc_ref, h_ref, p_ref, q_ref) = rest
    else:
        (h_ref,) = rest
    rc = rc_ref[...]
    mean = sm_ref[...] * rc
    var = sq_ref[...] * rc - mean * mean
    emb = jnp.concatenate([mn_ref[...], mx_ref[...], mean, var], axis=1)
    u = _relu(jnp.dot(emb, wu1_ref[...], preferred_element_type=jnp.float32)
              + bu1_ref[...])
    u = _relu(jnp.dot(u, wu2_ref[...], preferred_element_type=jnp.float32)
              + bu2_ref[...])
    h = u * BNC
    h_ref[...] = h
    if make_tables:
        xc = xc_ref[...]
        p_ref[...] = (
            jnp.dot(h, a_ref[...], preferred_element_type=jnp.float32)
            + pb_ref[...] - xc
        )
        q_ref[...] = (
            jnp.dot(h, bw_ref[...], preferred_element_type=jnp.float32) + xc
        )


def _node_update(mn, mx, sm, sq, rcnt, wu1, bu1, wu2, bu2,
                 nxt=None):
    n = mn.shape[0]
    bn = 1000
    row = lambda i: (i, 0)
    full = lambda i: (0, 0)
    in_specs = [
        pl.BlockSpec((bn, MSG), row),
        pl.BlockSpec((bn, MSG), row),
        pl.BlockSpec((bn, MSG), row),
        pl.BlockSpec((bn, MSG), row),
        pl.BlockSpec((bn, 1), row),
        pl.BlockSpec((512, MSG), full),
        pl.BlockSpec((1, MSG), full),
        pl.BlockSpec((MSG, MSG), full),
        pl.BlockSpec((1, MSG), full),
    ]
    args = [mn, mx, sm, sq, rcnt, wu1, bu1, wu2, bu2]
    make_tables = nxt is not None
    if make_tables:
        a, bw, pb, xc = nxt
        in_specs += [
            pl.BlockSpec((MSG, MSG), full),
            pl.BlockSpec((MSG, MSG), full),
            pl.BlockSpec((1, MSG), full),
            pl.BlockSpec((bn, MSG), row),
        ]
        args += [a, bw, pb, xc]
        out_specs = [pl.BlockSpec((bn, MSG), row)] * 3
        out_shape = [jax.ShapeDtypeStruct((n, MSG), jnp.float32)] * 3
    else:
        out_specs = pl.BlockSpec((bn, MSG), row)
        out_shape = jax.ShapeDtypeStruct((n, MSG), jnp.float32)
    return pl.pallas_call(
        functools.partial(_kd_body, make_tables),
        grid=(n // bn,),
        in_specs=in_specs,
        out_specs=out_specs,
        out_shape=out_shape,
    )(*args)


def _kf_body(mx_ref, mn_ref, sm_ref, rg_ref, d1_ref, db1_ref, d2_ref, db2_ref,
             d3_ref, db3_ref, hw_ref, hb_ref, o_ref):
    psum = sm_ref[...]
    g = jnp.concatenate([mx_ref[...], psum * rg_ref[...], psum, mn_ref[...]],
                        axis=1)
    t = jnp.dot(g, d1_ref[...], preferred_element_type=jnp.float32) + db1_ref[...]
    t = jnp.dot(t, d2_ref[...], preferred_element_type=jnp.float32) + db2_ref[...]
    t = jnp.dot(t, d3_ref[...], preferred_element_type=jnp.float32) + db3_ref[...]
    cols = []
    for k in range(4):
        w0 = hw_ref[k, :192, :64]
        w1 = hw_ref[k, 192:256, :64]
        w2 = hw_ref[k, 256:320, 0:1]
        b0 = hb_ref[k, 0:1, :64]
        b1 = hb_ref[k, 1:2, :64]
        b2 = hb_ref[k, 2:3, 0:1]
        u = (jnp.dot(t, w0, preferred_element_type=jnp.float32) + b0) * BNC
        u = (jnp.dot(u, w1, preferred_element_type=jnp.float32) + b1) * BNC
        cols.append(jnp.dot(u, w2, preferred_element_type=jnp.float32) + b2)
    o = jnp.concatenate(cols, axis=1)
    d3 = o[:, 0:3]
    nrm = jnp.sqrt(jnp.sum(d3 * d3, axis=1, keepdims=True))
    pos = nrm > 0.0
    dirv = jnp.where(pos, d3 / jnp.where(pos, nrm, 1.0), 0.0)
    o_ref[...] = jnp.concatenate([dirv, o[:, 3:4]], axis=1)


def _decode(gmx, gmn, gsm, rgcnt, d1, db1, d2, db2, d3, db3, hw, hb):
    full = lambda: (0, 0)
    return pl.pallas_call(
        _kf_body,
        out_shape=jax.ShapeDtypeStruct((NGRAPH, 4), jnp.float32),
    )(gmx, gmn, gsm, rgcnt, d1, db1, d2, db2, d3, db3, hw, hb)


def _prep(params):
    inv_s = 1.0 / _SCALE
    tcorr = -(_TRANS * inv_s)

    mp = params["mp"]
    A0 = mp[0]["Wm1"][0:NIN]
    B0 = mp[0]["Wm1"][NIN:2 * NIN]
    C0 = mp[0]["Wm1"][2 * NIN:3 * NIN]
    PW0 = A0 - BNC * C0
    QW0 = B0 + BNC * C0
    CW1 = BNC * mp[1]["Wm1"][2 * MSG:2 * MSG + NIN]
    CW2 = BNC * mp[2]["Wm1"][2 * MSG:2 * MSG + NIN]
    w0c = jnp.concatenate([PW0, QW0, CW1, CW2], axis=1) * inv_s[:, None]
    b0c = jnp.concatenate([
        mp[0]["bm1"] + tcorr @ PW0,
        tcorr @ QW0,
        tcorr @ CW1,
        tcorr @ CW2,
    ])[None, :]

    layers = []
    for l in range(3):
        lp = mp[l]
        d = {
            "Wm2": lp["Wm2"], "bm2": lp["bm2"][None, :],
            "Wu1": lp["Wu1"], "bu1": lp["bu1"][None, :],
            "Wu2": lp["Wu2"], "bu2": lp["bu2"][None, :],
        }
        if l < 2:
            nxt = mp[l + 1]
            d["A"] = nxt["Wm1"][0:MSG]
            d["Bw"] = nxt["Wm1"][MSG:2 * MSG]
            d["pb"] = nxt["bm1"][None, :]
        layers.append(d)

    dec = params["dec"]
    d1, db1 = dec[0][0] * BNC, dec[0][1][None, :] * BNC
    d2, db2 = dec[1][0] * BNC, dec[1][1][None, :] * BNC
    d3, db3 = dec[2][0] * BNC, dec[2][1][None, :] * BNC

    hws, hbs = [], []
    for sp in params["split"]:
        (W0, b0), (W1, b1), (W2, b2) = sp
        w = jnp.zeros((320, 64), jnp.float32)
        w = w.at[0:192, :].set(W0)
        w = w.at[192:256, :].set(W1)
        w = w.at[256:320, 0:1].set(W2)
        b = jnp.zeros((3, 64), jnp.float32)
        b = b.at[0, :].set(b0)
        b = b.at[1, :].set(b1)
        b = b.at[2, 0].set(b2[0])
        hws.append(w)
        hbs.append(b)
    hw = jnp.stack(hws)
    hb = jnp.stack(hbs)

    return dict(w0c=w0c, b0c=b0c, layers=layers,
                d1=d1, db1=db1, d2=d2, db2=db2, d3=d3, db3=db3,
                hw=hw, hb=hb)


def kernel(x, edge_index, graph_ids, params):
    pp = _prep(params)
    idx_i = edge_index[:, 0].astype(jnp.int32)
    idx_j = edge_index[:, 1].astype(jnp.int32)
    zpad = jnp.zeros((EP - NE,), jnp.int32)
    iip = jnp.concatenate([idx_i, zpad])
    jjp = jnp.concatenate([idx_j, zpad])
    x_pad = jnp.pad(x, ((0, NP - NN), (0, 0)))

    t0 = _init_tables(x_pad, pp["w0c"], pp["b0c"])
    p = t0[:, 0:128]
    q = t0[:, 128:256]
    xc1 = t0[:NN, 256:384]
    xc2 = t0[:NN, 384:512]
    xcs = [None, xc1, xc2]

    cnt = jax.ops.segment_sum(jnp.ones((NE,), jnp.float32), idx_i,
                              num_segments=NN)
    rcnt = (1.0 / cnt)[:, None]

    h = None
    for l in range(3):
        lw = pp["layers"][l]
        pg, qg = _sc_gather(p, q, iip, jjp)
        m = _edge_mlp(pg, qg, lw["Wm2"], lw["bm2"])
        m = m[:NE]
        mn = jax.ops.segment_min(m, idx_i, num_segments=NN)
        mx = jax.ops.segment_max(m, idx_i, num_segments=NN)
        sm = jax.ops.segment_sum(m, idx_i, num_segments=NN)
        sq = jax.ops.segment_sum(m * m, idx_i, num_segments=NN)
        if l < 2:
            nxt = (lw["A"], lw["Bw"], lw["pb"], xcs[l + 1])
            h, p, q = _node_update(mn, mx, sm, sq, rcnt, lw["Wu1"], lw["bu1"],
                                   lw["Wu2"], lw["bu2"], nxt)
            p = jnp.pad(p, ((0, NP - NN), (0, 0)))
            q = jnp.pad(q, ((0, NP - NN), (0, 0)))
        else:
            h = _node_update(mn, mx, sm, sq, rcnt, lw["Wu1"], lw["bu1"],
                             lw["Wu2"], lw["bu2"])

    gmx = jax.ops.segment_max(h, graph_ids, num_segments=NGRAPH)
    gmn = jax.ops.segment_min(h, graph_ids, num_segments=NGRAPH)
    gsm = jax.ops.segment_sum(h, graph_ids, num_segments=NGRAPH)
    gcnt = jax.ops.segment_sum(jnp.ones((NN,), jnp.float32), graph_ids,
                               num_segments=NGRAPH)
    rgcnt = (1.0 / gcnt)[:, None]
    return _decode(gmx, gmn, gsm, rgcnt, pp["d1"], pp["db1"], pp["d2"],
                   pp["db2"], pp["d3"], pp["db3"], pp["hw"], pp["hb"])


# speedup vs baseline: 2.6693x; 2.3601x over previous
import dataclasses
import functools

import jax
import jax.numpy as jnp
import numpy as np
from jax.experimental import pallas as pl
from jax.experimental.pallas import tpu as pltpu
from jax.experimental.pallas import tpu_sc as plsc

NN = 10000
NE = 160000
NP = 10240
EP = 163840
NIN = 5
MSG = 128
NGRAPH = 64
HS = 64
BNC = float(1.0 / np.sqrt(1.0 + 1e-3))
_TRANS = np.array([0.0, 0.0, -200.0, 10000.0, 0.0], dtype=np.float32)
_SCALE = np.array([100.0, 100.0, 100.0, 2500.0, 0.25], dtype=np.float32)


def _relu(v):
    return jnp.maximum(v, 0.0)


def _k0_body(x_ref, w_ref, b_ref, o_ref):
    o_ref[...] = (
        jnp.dot(x_ref[...], w_ref[...], preferred_element_type=jnp.float32)
        + b_ref[...]
    )


def _init_tables(x, w0c, b0c):
    n = x.shape[0]
    bn = 1024
    return pl.pallas_call(
        _k0_body,
        grid=(n // bn,),
        in_specs=[
            pl.BlockSpec((bn, NIN), lambda i: (i, 0)),
            pl.BlockSpec((NIN, 512), lambda i: (0, 0)),
            pl.BlockSpec((1, 512), lambda i: (0, 0)),
        ],
        out_specs=pl.BlockSpec((bn, 512), lambda i: (i, 0)),
        out_shape=jax.ShapeDtypeStruct((n, 512), jnp.float32),
    )(x, w0c, b0c)


GW = 128


def _sc_gather(p, q, ii, jj):
    mesh = plsc.VectorSubcoreMesh(core_axis_name="c", subcore_axis_name="s")
    ii2 = ii.reshape(1, EP)
    jj2 = jj.reshape(1, EP)

    @functools.partial(
        pl.kernel,
        out_type=[jax.ShapeDtypeStruct((EP, MSG), jnp.float32),
                  jax.ShapeDtypeStruct((EP, MSG), jnp.float32)],
        mesh=mesh,
    )
    def k(p_hbm, q_hbm, ii_hbm, jj_hbm, pg_hbm, qg_hbm):
        def body(ii_v, jj_v, po_v, qo_v):
            pltpu.sync_copy(p_hbm.at[ii_v.at[0]], po_v)
            pltpu.sync_copy(q_hbm.at[jj_v.at[0]], qo_v)

        pltpu.emit_pipeline(
            body,
            grid=(EP // GW,),
            in_specs=[pl.BlockSpec((1, GW), lambda i: (0, i)),
                      pl.BlockSpec((1, GW), lambda i: (0, i))],
            out_specs=[pl.BlockSpec((GW, MSG), lambda i: (i, 0)),
                       pl.BlockSpec((GW, MSG), lambda i: (i, 0))],
            core_axis_name=("c", "s"),
            dimension_semantics=(pltpu.PARALLEL,),
        )(ii_hbm, jj_hbm, pg_hbm, qg_hbm)

    return k(p, q, ii2, jj2)


RW = 128


def _sc_compiler_params():
    cp = pltpu.CompilerParams()
    if "needs_layout_passes" in pltpu.CompilerParams.__dataclass_fields__:
        cp = dataclasses.replace(cp, needs_layout_passes=False)
    return cp


def _acc_init():
    return (tuple(jnp.full((16,), 3e38, jnp.float32) for _ in range(8))
            + tuple(jnp.full((16,), -3e38, jnp.float32) for _ in range(8))
            + tuple(jnp.zeros((16,), jnp.float32) for _ in range(16)))


def _acc_row(buf, r, accs):
    out = []
    for j in range(8):
        v = buf[r, pl.ds(j * 16, 16)]
        out.append(jnp.minimum(accs[j], v))
    for j in range(8):
        v = buf[r, pl.ds(j * 16, 16)]
        out.append(jnp.maximum(accs[8 + j], v))
    for j in range(8):
        v = buf[r, pl.ds(j * 16, 16)]
        out.append(accs[16 + j] + v)
    for j in range(8):
        v = buf[r, pl.ds(j * 16, 16)]
        out.append(accs[24 + j] + v * v)
    return tuple(out)


def _make_sc_reduce(nseg_pad, npw, ngrp, in_rows, rw=RW):
    gsz = npw // ngrp
    load_all = npw % 8 != 0
    olen = nseg_pad + 24 if load_all else npw + 24
    aligned_out = gsz % 8 == 0
    stg_rows = gsz if aligned_out else 8
    out_rows = nseg_pad if aligned_out else 32 * 8
    mesh = plsc.VectorSubcoreMesh(core_axis_name="c", subcore_axis_name="s")

    def _og(ref, i):
        return ref[pl.ds(i, 16)][0]

    @functools.partial(
        pl.kernel,
        out_type=jax.ShapeDtypeStruct((out_rows, 512), jnp.float32),
        mesh=mesh,
        compiler_params=_sc_compiler_params(),
        scratch_types=[
            pltpu.VMEM((olen,), jnp.int32),
            pltpu.VMEM((rw, 128), jnp.float32),
            pltpu.VMEM((rw, 128), jnp.float32),
            pltpu.VMEM((stg_rows, 512), jnp.float32),
            pltpu.SemaphoreType.DMA,
            pltpu.SemaphoreType.DMA,
        ],
    )
    def k(m_hbm, offs_hbm, emb_hbm, offs_v, buf0, buf1, stg, sem0, sem1):
        wid = jax.lax.axis_index("s") * 2 + jax.lax.axis_index("c")
        nb = wid * npw
        if load_all:
            pltpu.sync_copy(offs_hbm.at[pl.ds(0, olen)], offs_v)
            ob = 0
        else:
            pltpu.sync_copy(offs_hbm.at[pl.ds(nb, olen)], offs_v)
            ob = None

        def flush(nloc, accs):
            for j in range(32):
                stg[nloc, pl.ds(j * 16, 16)] = accs[j]

        for grp in range(ngrp):
            gb = grp * gsz
            obase = nb + gb if load_all else gb
            s0 = _og(offs_v, obase)
            s1 = _og(offs_v, obase + gsz)
            s0a = jnp.bitwise_and(s0, -8)
            nwin = (s1 - s0a + (rw - 1)) // rw
            nwin2 = jnp.bitwise_and(nwin + 1, -2)

            def issue(wi, buf, sem):
                pltpu.make_async_copy(
                    m_hbm.at[pl.ds(pl.multiple_of(s0a + wi * rw, 8), rw)],
                    buf, sem).start()

            @pl.when(nwin > 0)
            def _():
                issue(0, buf0, sem0)
                issue(1, buf1, sem1)

            def process(buf, sem, wi, carry):
                pltpu.make_async_copy(
                    m_hbm.at[pl.ds(0, rw)], buf, sem).wait()
                wb = s0a + wi * rw
                wend = jnp.minimum(wb + rw, s1)

                nend = jnp.int32(0)
                for t in range(0, gsz, 16):
                    v = offs_v[pl.ds(obase + 1 + t, 16)]
                    msk = v <= wend
                    if gsz - t < 16:
                        msk = jnp.logical_and(
                            msk, jax.lax.iota(jnp.int32, 16) < (gsz - t))
                    nend = nend + plsc.all_reduce_population_count(msk)[0]

                def fbody(_, c):
                    ni = c[0]
                    rs = jnp.maximum(_og(offs_v, obase + (ni - gb)), wb)
                    re_ = _og(offs_v, obase + (ni - gb) + 1)
                    accs = jax.lax.fori_loop(
                        rs - wb, re_ - wb,
                        lambda r, a: _acc_row(buf, r, a), tuple(c[1:]))
                    flush(ni - gb, accs)
                    return (ni + 1,) + _acc_init()

                c = jax.lax.fori_loop(0, nend - (carry[0] - gb), fbody, carry)
                ni = c[0]
                rs = jnp.maximum(_og(offs_v, obase + (ni - gb)), wb)
                accs = jax.lax.fori_loop(
                    rs - wb, wend - wb,
                    lambda r, a: _acc_row(buf, r, a), tuple(c[1:]))

                @pl.when(wi + 2 < nwin2)
                def _():
                    issue(wi + 2, buf, sem)

                return (ni,) + accs

            def pair_body(k, carry):
                carry = process(buf0, sem0, 2 * k, carry)
                return process(buf1, sem1, 2 * k + 1, carry)

            carry = (gb,) + _acc_init()
            jax.lax.fori_loop(0, nwin2 // 2, pair_body, carry)
            srow = nb + gb if aligned_out else wid * 8
            pltpu.sync_copy(
                stg, emb_hbm.at[pl.ds(pl.multiple_of(srow, 8), stg_rows)])

    return k


def _kb_body(p_ref, q_ref, w_ref, b_ref, o_ref):
    m1 = _relu(p_ref[...] + q_ref[...])
    o_ref[...] = _relu(
        jnp.dot(m1, w_ref[...], preferred_element_type=jnp.float32) + b_ref[...]
    )


def _edge_mlp(pg, qg, w2, b2):
    e = pg.shape[0]
    be = 2048
    return pl.pallas_call(
        _kb_body,
        grid=(e // be,),
        in_specs=[
            pl.BlockSpec((be, MSG), lambda i: (i, 0)),
            pl.BlockSpec((be, MSG), lambda i: (i, 0)),
            pl.BlockSpec((MSG, MSG), lambda i: (0, 0)),
            pl.BlockSpec((1, MSG), lambda i: (0, 0)),
        ],
        out_specs=pl.BlockSpec((be, MSG), lambda i: (i, 0)),
        out_shape=jax.ShapeDtypeStruct((e, MSG), jnp.float32),
    )(pg, qg, w2, b2)


def _kd_body(make_tables, emb_ref, rc_ref, wu1_ref,
             bu1_ref, wu2_ref, bu2_ref, *rest):
    if make_tables:
        (a_ref, bw_ref, pb_ref, xc_ref, h_ref, p_ref, q_ref) = rest
    else:
        (h_ref,) = rest
    rc = rc_ref[...]
    raw = emb_ref[...]
    mean = raw[:, 256:384] * rc
    var = raw[:, 384:512] * rc - mean * mean
    emb = jnp.concatenate([raw[:, 0:256], mean, var], axis=1)
    u = _relu(jnp.dot(emb, wu1_ref[...], preferred_element_type=jnp.float32)
              + bu1_ref[...])
    u = _relu(jnp.dot(u, wu2_ref[...], preferred_element_type=jnp.float32)
              + bu2_ref[...])
    h = u * BNC
    h_ref[...] = h
    if make_tables:
        xc = xc_ref[...]
        p_ref[...] = (
            jnp.dot(h, a_ref[...], preferred_element_type=jnp.float32)
            + pb_ref[...] - xc
        )
        q_ref[...] = (
            jnp.dot(h, bw_ref[...], preferred_element_type=jnp.float32) + xc
        )


def _node_update(emb, rcnt, wu1, bu1, wu2, bu2, nxt=None):
    n = emb.shape[0]
    bn = 1024
    row = lambda i: (i, 0)
    full = lambda i: (0, 0)
    in_specs = [
        pl.BlockSpec((bn, 512), row),
        pl.BlockSpec((bn, 1), row),
        pl.BlockSpec((512, MSG), full),
        pl.BlockSpec((1, MSG), full),
        pl.BlockSpec((MSG, MSG), full),
        pl.BlockSpec((1, MSG), full),
    ]
    args = [emb, rcnt, wu1, bu1, wu2, bu2]
    make_tables = nxt is not None
    if make_tables:
        a, bw, pb, xc = nxt
        in_specs += [
            pl.BlockSpec((MSG, MSG), full),
            pl.BlockSpec((MSG, MSG), full),
            pl.BlockSpec((1, MSG), full),
            pl.BlockSpec((bn, MSG), row),
        ]
        args += [a, bw, pb, xc]
        out_specs = [pl.BlockSpec((bn, MSG), row)] * 3
        out_shape = [jax.ShapeDtypeStruct((n, MSG), jnp.float32)] * 3
    else:
        out_specs = pl.BlockSpec((bn, MSG), row)
        out_shape = jax.ShapeDtypeStruct((n, MSG), jnp.float32)
    return pl.pallas_call(
        functools.partial(_kd_body, make_tables),
        grid=(n // bn,),
        in_specs=in_specs,
        out_specs=out_specs,
        out_shape=out_shape,
    )(*args)


def _kf_body(pool_ref, rg_ref, d1_ref, db1_ref, d2_ref, db2_ref,
             d3_ref, db3_ref, hw_ref, hb_ref, o_ref):
    raw = pool_ref[...]
    psum = raw[:, 256:384]
    g = jnp.concatenate([raw[:, 128:256], psum * rg_ref[...], psum,
                         raw[:, 0:128]], axis=1)
    t = jnp.dot(g, d1_ref[...], preferred_element_type=jnp.float32) + db1_ref[...]
    t = jnp.dot(t, d2_ref[...], preferred_element_type=jnp.float32) + db2_ref[...]
    t = jnp.dot(t, d3_ref[...], preferred_element_type=jnp.float32) + db3_ref[...]
    cols = []
    for k in range(4):
        w0 = hw_ref[k, :192, :64]
        w1 = hw_ref[k, 192:256, :64]
        w2 = hw_ref[k, 256:320, 0:1]
        b0 = hb_ref[k, 0:1, :64]
        b1 = hb_ref[k, 1:2, :64]
        b2 = hb_ref[k, 2:3, 0:1]
        u = (jnp.dot(t, w0, preferred_element_type=jnp.float32) + b0) * BNC
        u = (jnp.dot(u, w1, preferred_element_type=jnp.float32) + b1) * BNC
        cols.append(jnp.dot(u, w2, preferred_element_type=jnp.float32) + b2)
    o = jnp.concatenate(cols, axis=1)
    d3 = o[:, 0:3]
    nrm = jnp.sqrt(jnp.sum(d3 * d3, axis=1, keepdims=True))
    pos = nrm > 0.0
    dirv = jnp.where(pos, d3 / jnp.where(pos, nrm, 1.0), 0.0)
    o_ref[...] = jnp.concatenate([dirv, o[:, 3:4]], axis=1)


def _decode(pool, rgcnt, d1, db1, d2, db2, d3, db3, hw, hb):
    return pl.pallas_call(
        _kf_body,
        out_shape=jax.ShapeDtypeStruct((NGRAPH, 4), jnp.float32),
    )(pool, rgcnt, d1, db1, d2, db2, d3, db3, hw, hb)


def _prep(params):
    inv_s = 1.0 / _SCALE
    tcorr = -(_TRANS * inv_s)

    mp = params["mp"]
    A0 = mp[0]["Wm1"][0:NIN]
    B0 = mp[0]["Wm1"][NIN:2 * NIN]
    C0 = mp[0]["Wm1"][2 * NIN:3 * NIN]
    PW0 = A0 - BNC * C0
    QW0 = B0 + BNC * C0
    CW1 = BNC * mp[1]["Wm1"][2 * MSG:2 * MSG + NIN]
    CW2 = BNC * mp[2]["Wm1"][2 * MSG:2 * MSG + NIN]
    w0c = jnp.concatenate([PW0, QW0, CW1, CW2], axis=1) * inv_s[:, None]
    b0c = jnp.concatenate([
        mp[0]["bm1"] + tcorr @ PW0,
        tcorr @ QW0,
        tcorr @ CW1,
        tcorr @ CW2,
    ])[None, :]

    layers = []
    for l in range(3):
        lp = mp[l]
        d = {
            "Wm2": lp["Wm2"], "bm2": lp["bm2"][None, :],
            "Wu1": lp["Wu1"], "bu1": lp["bu1"][None, :],
            "Wu2": lp["Wu2"], "bu2": lp["bu2"][None, :],
        }
        if l < 2:
            nxt = mp[l + 1]
            d["A"] = nxt["Wm1"][0:MSG]
            d["Bw"] = nxt["Wm1"][MSG:2 * MSG]
            d["pb"] = nxt["bm1"][None, :]
        layers.append(d)

    dec = params["dec"]
    d1, db1 = dec[0][0] * BNC, dec[0][1][None, :] * BNC
    d2, db2 = dec[1][0] * BNC, dec[1][1][None, :] * BNC
    d3, db3 = dec[2][0] * BNC, dec[2][1][None, :] * BNC

    hws, hbs = [], []
    for sp in params["split"]:
        (W0, b0), (W1, b1), (W2, b2) = sp
        w = jnp.zeros((320, 64), jnp.float32)
        w = w.at[0:192, :].set(W0)
        w = w.at[192:256, :].set(W1)
        w = w.at[256:320, 0:1].set(W2)
        b = jnp.zeros((3, 64), jnp.float32)
        b = b.at[0, :].set(b0)
        b = b.at[1, :].set(b1)
        b = b.at[2, 0].set(b2[0])
        hws.append(w)
        hbs.append(b)
    hw = jnp.stack(hws)
    hb = jnp.stack(hbs)

    return dict(w0c=w0c, b0c=b0c, layers=layers,
                d1=d1, db1=db1, d2=d2, db2=db2, d3=d3, db3=db3,
                hw=hw, hb=hb)


def kernel(x, edge_index, graph_ids, params):
    pp = _prep(params)
    idx_i = edge_index[:, 0].astype(jnp.int32)
    idx_j = edge_index[:, 1].astype(jnp.int32)
    zpad = jnp.zeros((EP - NE,), jnp.int32)
    iip = jnp.concatenate([idx_i, zpad])
    jjp = jnp.concatenate([idx_j, zpad])
    x_pad = jnp.pad(x, ((0, NP - NN), (0, 0)))

    t0 = _init_tables(x_pad, pp["w0c"], pp["b0c"])
    p = t0[:, 0:128]
    q = t0[:, 128:256]
    xcs = [None, t0[:, 256:384], t0[:, 384:512]]

    offs = jnp.searchsorted(idx_i, jnp.arange(NP + 24, dtype=jnp.int32),
                            side="left").astype(jnp.int32)
    goffs = jnp.searchsorted(graph_ids.astype(jnp.int32),
                             jnp.arange(NGRAPH + 24, dtype=jnp.int32),
                             side="left").astype(jnp.int32)
    cnt = (offs[1:NP + 1] - offs[:NP]).astype(jnp.float32)
    rcnt = (1.0 / jnp.maximum(cnt, 1.0))[:, None]
    gcnt = (goffs[1:NGRAPH + 1] - goffs[:NGRAPH]).astype(jnp.float32)
    rgcnt = (1.0 / jnp.maximum(gcnt, 1.0))[:, None]

    seg_reduce = _make_sc_reduce(NP, 320, 4, EP)
    pool_reduce = _make_sc_reduce(NGRAPH, 2, 1, NP, rw=64)

    h = None
    for l in range(3):
        lw = pp["layers"][l]
        pg, qg = _sc_gather(p, q, iip, jjp)
        m = _edge_mlp(pg, qg, lw["Wm2"], lw["bm2"])
        emb = seg_reduce(m, offs)
        if l < 2:
            nxt = (lw["A"], lw["Bw"], lw["pb"], xcs[l + 1])
            h, p, q = _node_update(emb, rcnt, lw["Wu1"], lw["bu1"],
                                   lw["Wu2"], lw["bu2"], nxt)
        else:
            h = _node_update(emb, rcnt, lw["Wu1"], lw["bu1"],
                             lw["Wu2"], lw["bu2"])

    pool_raw = pool_reduce(h, goffs)
    pool = pool_raw.reshape(32, 8, 512)[:, 0:2, :].reshape(NGRAPH, 512)
    return _decode(pool, rgcnt, pp["d1"], pp["db1"], pp["d2"],
                   pp["db2"], pp["d3"], pp["db3"], pp["hw"], pp["hb"])


# speedup vs baseline: 3.3946x; 1.2717x over previous
import dataclasses
import functools

import jax
import jax.numpy as jnp
import numpy as np
from jax.experimental import pallas as pl
from jax.experimental.pallas import tpu as pltpu
from jax.experimental.pallas import tpu_sc as plsc

NN = 10000
NE = 160000
NP = 10240
EP = 163840
NIN = 5
MSG = 128
NGRAPH = 64
HS = 64
BNC = float(1.0 / np.sqrt(1.0 + 1e-3))
_TRANS = np.array([0.0, 0.0, -200.0, 10000.0, 0.0], dtype=np.float32)
_SCALE = np.array([100.0, 100.0, 100.0, 2500.0, 0.25], dtype=np.float32)


def _relu(v):
    return jnp.maximum(v, 0.0)


def _k0_body(x_ref, w_ref, b_ref, o_ref):
    o_ref[...] = (
        jnp.dot(x_ref[...], w_ref[...], preferred_element_type=jnp.float32)
        + b_ref[...]
    )


def _init_tables(x, w0c, b0c):
    n = x.shape[0]
    bn = 1024
    return pl.pallas_call(
        _k0_body,
        grid=(n // bn,),
        in_specs=[
            pl.BlockSpec((bn, NIN), lambda i: (i, 0)),
            pl.BlockSpec((NIN, 512), lambda i: (0, 0)),
            pl.BlockSpec((1, 512), lambda i: (0, 0)),
        ],
        out_specs=pl.BlockSpec((bn, 512), lambda i: (i, 0)),
        out_shape=jax.ShapeDtypeStruct((n, 512), jnp.float32),
    )(x, w0c, b0c)


GW = 128


def _sc_gather(p, q, ii, jj):
    mesh = plsc.VectorSubcoreMesh(core_axis_name="c", subcore_axis_name="s")
    ii2 = ii.reshape(1, EP)
    jj2 = jj.reshape(1, EP)

    @functools.partial(
        pl.kernel,
        out_type=[jax.ShapeDtypeStruct((EP, MSG), jnp.float32),
                  jax.ShapeDtypeStruct((EP, MSG), jnp.float32)],
        mesh=mesh,
        scratch_types=[pltpu.SemaphoreType.DMA, pltpu.SemaphoreType.DMA],
    )
    def k(p_hbm, q_hbm, ii_hbm, jj_hbm, pg_hbm, qg_hbm, sem_a, sem_b):
        def body(ii_v, jj_v, po_v, qo_v):
            c1 = pltpu.async_copy(p_hbm.at[ii_v.at[0]], po_v, sem_a)
            c2 = pltpu.async_copy(q_hbm.at[jj_v.at[0]], qo_v, sem_b)
            c1.wait()
            c2.wait()

        pltpu.emit_pipeline(
            body,
            grid=(EP // GW,),
            in_specs=[pl.BlockSpec((1, GW), lambda i: (0, i)),
                      pl.BlockSpec((1, GW), lambda i: (0, i))],
            out_specs=[pl.BlockSpec((GW, MSG), lambda i: (i, 0)),
                       pl.BlockSpec((GW, MSG), lambda i: (i, 0))],
            core_axis_name=("c", "s"),
            dimension_semantics=(pltpu.PARALLEL,),
        )(ii_hbm, jj_hbm, pg_hbm, qg_hbm)

    return k(p, q, ii2, jj2)


RW = 128


def _sc_compiler_params():
    cp = pltpu.CompilerParams()
    if "needs_layout_passes" in pltpu.CompilerParams.__dataclass_fields__:
        cp = dataclasses.replace(cp, needs_layout_passes=False)
    return cp


def _acc_init():
    return (tuple(jnp.full((16,), 3e38, jnp.float32) for _ in range(8))
            + tuple(jnp.full((16,), -3e38, jnp.float32) for _ in range(8))
            + tuple(jnp.zeros((16,), jnp.float32) for _ in range(16)))


def _acc_row(buf, r, accs):
    out = []
    for j in range(8):
        v = buf[r, pl.ds(j * 16, 16)]
        out.append(jnp.minimum(accs[j], v))
    for j in range(8):
        v = buf[r, pl.ds(j * 16, 16)]
        out.append(jnp.maximum(accs[8 + j], v))
    for j in range(8):
        v = buf[r, pl.ds(j * 16, 16)]
        out.append(accs[16 + j] + v)
    for j in range(8):
        v = buf[r, pl.ds(j * 16, 16)]
        out.append(accs[24 + j] + v * v)
    return tuple(out)


def _make_sc_reduce(nseg_pad, npw, ngrp, in_rows, rw=RW):
    gsz = npw // ngrp
    load_all = npw % 8 != 0
    olen = nseg_pad + 24 if load_all else npw + 24
    aligned_out = gsz % 8 == 0
    stg_rows = gsz if aligned_out else 8
    out_rows = nseg_pad if aligned_out else 32 * 8
    mesh = plsc.VectorSubcoreMesh(core_axis_name="c", subcore_axis_name="s")

    def _og(ref, i):
        return ref[pl.ds(i, 16)][0]

    @functools.partial(
        pl.kernel,
        out_type=jax.ShapeDtypeStruct((out_rows, 512), jnp.float32),
        mesh=mesh,
        compiler_params=_sc_compiler_params(),
        scratch_types=[
            pltpu.VMEM((olen,), jnp.int32),
            pltpu.VMEM((rw, 128), jnp.float32),
            pltpu.VMEM((rw, 128), jnp.float32),
            pltpu.VMEM((stg_rows, 512), jnp.float32),
            pltpu.SemaphoreType.DMA,
            pltpu.SemaphoreType.DMA,
        ],
    )
    def k(m_hbm, offs_hbm, emb_hbm, offs_v, buf0, buf1, stg, sem0, sem1):
        wid = jax.lax.axis_index("s") * 2 + jax.lax.axis_index("c")
        nb = wid * npw
        if load_all:
            pltpu.sync_copy(offs_hbm.at[pl.ds(0, olen)], offs_v)
            ob = 0
        else:
            pltpu.sync_copy(offs_hbm.at[pl.ds(nb, olen)], offs_v)
            ob = None

        def flush(nloc, accs):
            for j in range(32):
                stg[nloc, pl.ds(j * 16, 16)] = accs[j]

        for grp in range(ngrp):
            gb = grp * gsz
            obase = nb + gb if load_all else gb
            s0 = _og(offs_v, obase)
            s1 = _og(offs_v, obase + gsz)
            s0a = jnp.bitwise_and(s0, -8)
            nwin = (s1 - s0a + (rw - 1)) // rw
            nwin2 = jnp.bitwise_and(nwin + 1, -2)

            def issue(wi, buf, sem):
                pltpu.make_async_copy(
                    m_hbm.at[pl.ds(pl.multiple_of(s0a + wi * rw, 8), rw)],
                    buf, sem).start()

            @pl.when(nwin > 0)
            def _():
                issue(0, buf0, sem0)
                issue(1, buf1, sem1)

            def process(buf, sem, wi, carry):
                pltpu.make_async_copy(
                    m_hbm.at[pl.ds(0, rw)], buf, sem).wait()
                wb = s0a + wi * rw
                wend = jnp.minimum(wb + rw, s1)

                nend = jnp.int32(0)
                for t in range(0, gsz, 16):
                    v = offs_v[pl.ds(obase + 1 + t, 16)]
                    msk = v <= wend
                    if gsz - t < 16:
                        msk = jnp.logical_and(
                            msk, jax.lax.iota(jnp.int32, 16) < (gsz - t))
                    nend = nend + plsc.all_reduce_population_count(msk)[0]

                def fbody(_, c):
                    ni = c[0]
                    rs = jnp.maximum(_og(offs_v, obase + (ni - gb)), wb)
                    re_ = _og(offs_v, obase + (ni - gb) + 1)
                    accs = jax.lax.fori_loop(
                        rs - wb, re_ - wb,
                        lambda r, a: _acc_row(buf, r, a), tuple(c[1:]))
                    flush(ni - gb, accs)
                    return (ni + 1,) + _acc_init()

                c = jax.lax.fori_loop(0, nend - (carry[0] - gb), fbody, carry)
                ni = c[0]
                rs = jnp.maximum(_og(offs_v, obase + (ni - gb)), wb)
                accs = jax.lax.fori_loop(
                    rs - wb, wend - wb,
                    lambda r, a: _acc_row(buf, r, a), tuple(c[1:]))

                @pl.when(wi + 2 < nwin2)
                def _():
                    issue(wi + 2, buf, sem)

                return (ni,) + accs

            def pair_body(k, carry):
                carry = process(buf0, sem0, 2 * k, carry)
                return process(buf1, sem1, 2 * k + 1, carry)

            carry = (gb,) + _acc_init()
            jax.lax.fori_loop(0, nwin2 // 2, pair_body, carry)
            srow = nb + gb if aligned_out else wid * 8
            pltpu.sync_copy(
                stg, emb_hbm.at[pl.ds(pl.multiple_of(srow, 8), stg_rows)])

    return k


def _kb_body(p_ref, q_ref, w_ref, b_ref, o_ref):
    m1 = _relu(p_ref[...] + q_ref[...])
    o_ref[...] = _relu(
        jnp.dot(m1, w_ref[...], preferred_element_type=jnp.float32) + b_ref[...]
    )


def _edge_mlp(pg, qg, w2, b2):
    e = pg.shape[0]
    be = 2048
    return pl.pallas_call(
        _kb_body,
        grid=(e // be,),
        in_specs=[
            pl.BlockSpec((be, MSG), lambda i: (i, 0)),
            pl.BlockSpec((be, MSG), lambda i: (i, 0)),
            pl.BlockSpec((MSG, MSG), lambda i: (0, 0)),
            pl.BlockSpec((1, MSG), lambda i: (0, 0)),
        ],
        out_specs=pl.BlockSpec((be, MSG), lambda i: (i, 0)),
        out_shape=jax.ShapeDtypeStruct((e, MSG), jnp.float32),
    )(pg, qg, w2, b2)


def _kd_body(make_tables, emb_ref, rc_ref, wu1_ref,
             bu1_ref, wu2_ref, bu2_ref, *rest):
    if make_tables:
        (a_ref, bw_ref, pb_ref, xc_ref, h_ref, p_ref, q_ref) = rest
    else:
        (h_ref,) = rest
    rc = rc_ref[...]
    raw = emb_ref[...]
    mean = raw[:, 256:384] * rc
    var = raw[:, 384:512] * rc - mean * mean
    emb = jnp.concatenate([raw[:, 0:256], mean, var], axis=1)
    u = _relu(jnp.dot(emb, wu1_ref[...], preferred_element_type=jnp.float32)
              + bu1_ref[...])
    u = _relu(jnp.dot(u, wu2_ref[...], preferred_element_type=jnp.float32)
              + bu2_ref[...])
    h = u * BNC
    h_ref[...] = h
    if make_tables:
        xc = xc_ref[...]
        p_ref[...] = (
            jnp.dot(h, a_ref[...], preferred_element_type=jnp.float32)
            + pb_ref[...] - xc
        )
        q_ref[...] = (
            jnp.dot(h, bw_ref[...], preferred_element_type=jnp.float32) + xc
        )


def _node_update(emb, rcnt, wu1, bu1, wu2, bu2, nxt=None):
    n = emb.shape[0]
    bn = 1024
    row = lambda i: (i, 0)
    full = lambda i: (0, 0)
    in_specs = [
        pl.BlockSpec((bn, 512), row),
        pl.BlockSpec((bn, 1), row),
        pl.BlockSpec((512, MSG), full),
        pl.BlockSpec((1, MSG), full),
        pl.BlockSpec((MSG, MSG), full),
        pl.BlockSpec((1, MSG), full),
    ]
    args = [emb, rcnt, wu1, bu1, wu2, bu2]
    make_tables = nxt is not None
    if make_tables:
        a, bw, pb, xc = nxt
        in_specs += [
            pl.BlockSpec((MSG, MSG), full),
            pl.BlockSpec((MSG, MSG), full),
            pl.BlockSpec((1, MSG), full),
            pl.BlockSpec((bn, MSG), row),
        ]
        args += [a, bw, pb, xc]
        out_specs = [pl.BlockSpec((bn, MSG), row)] * 3
        out_shape = [jax.ShapeDtypeStruct((n, MSG), jnp.float32)] * 3
    else:
        out_specs = pl.BlockSpec((bn, MSG), row)
        out_shape = jax.ShapeDtypeStruct((n, MSG), jnp.float32)
    return pl.pallas_call(
        functools.partial(_kd_body, make_tables),
        grid=(n // bn,),
        in_specs=in_specs,
        out_specs=out_specs,
        out_shape=out_shape,
    )(*args)


def _kf_body(pool_ref, rg_ref, d1_ref, db1_ref, d2_ref, db2_ref,
             d3_ref, db3_ref, hw_ref, hb_ref, o_ref):
    raw = pool_ref[...]
    psum = raw[:, 256:384]
    g = jnp.concatenate([raw[:, 128:256], psum * rg_ref[...], psum,
                         raw[:, 0:128]], axis=1)
    t = jnp.dot(g, d1_ref[...], preferred_element_type=jnp.float32) + db1_ref[...]
    t = jnp.dot(t, d2_ref[...], preferred_element_type=jnp.float32) + db2_ref[...]
    t = jnp.dot(t, d3_ref[...], preferred_element_type=jnp.float32) + db3_ref[...]
    cols = []
    for k in range(4):
        w0 = hw_ref[k, :192, :64]
        w1 = hw_ref[k, 192:256, :64]
        w2 = hw_ref[k, 256:320, 0:1]
        b0 = hb_ref[k, 0:1, :64]
        b1 = hb_ref[k, 1:2, :64]
        b2 = hb_ref[k, 2:3, 0:1]
        u = (jnp.dot(t, w0, preferred_element_type=jnp.float32) + b0) * BNC
        u = (jnp.dot(u, w1, preferred_element_type=jnp.float32) + b1) * BNC
        cols.append(jnp.dot(u, w2, preferred_element_type=jnp.float32) + b2)
    o = jnp.concatenate(cols, axis=1)
    d3 = o[:, 0:3]
    nrm = jnp.sqrt(jnp.sum(d3 * d3, axis=1, keepdims=True))
    pos = nrm > 0.0
    dirv = jnp.where(pos, d3 / jnp.where(pos, nrm, 1.0), 0.0)
    o_ref[...] = jnp.concatenate([dirv, o[:, 3:4]], axis=1)


def _decode(pool, rgcnt, d1, db1, d2, db2, d3, db3, hw, hb):
    return pl.pallas_call(
        _kf_body,
        out_shape=jax.ShapeDtypeStruct((NGRAPH, 4), jnp.float32),
    )(pool, rgcnt, d1, db1, d2, db2, d3, db3, hw, hb)


def _prep(params):
    inv_s = 1.0 / _SCALE
    tcorr = -(_TRANS * inv_s)

    mp = params["mp"]
    A0 = mp[0]["Wm1"][0:NIN]
    B0 = mp[0]["Wm1"][NIN:2 * NIN]
    C0 = mp[0]["Wm1"][2 * NIN:3 * NIN]
    PW0 = A0 - BNC * C0
    QW0 = B0 + BNC * C0
    CW1 = BNC * mp[1]["Wm1"][2 * MSG:2 * MSG + NIN]
    CW2 = BNC * mp[2]["Wm1"][2 * MSG:2 * MSG + NIN]
    w0c = jnp.concatenate([PW0, QW0, CW1, CW2], axis=1) * inv_s[:, None]
    b0c = jnp.concatenate([
        mp[0]["bm1"] + tcorr @ PW0,
        tcorr @ QW0,
        tcorr @ CW1,
        tcorr @ CW2,
    ])[None, :]

    layers = []
    for l in range(3):
        lp = mp[l]
        d = {
            "Wm2": lp["Wm2"], "bm2": lp["bm2"][None, :],
            "Wu1": lp["Wu1"], "bu1": lp["bu1"][None, :],
            "Wu2": lp["Wu2"], "bu2": lp["bu2"][None, :],
        }
        if l < 2:
            nxt = mp[l + 1]
            d["A"] = nxt["Wm1"][0:MSG]
            d["Bw"] = nxt["Wm1"][MSG:2 * MSG]
            d["pb"] = nxt["bm1"][None, :]
        layers.append(d)

    dec = params["dec"]
    d1, db1 = dec[0][0] * BNC, dec[0][1][None, :] * BNC
    d2, db2 = dec[1][0] * BNC, dec[1][1][None, :] * BNC
    d3, db3 = dec[2][0] * BNC, dec[2][1][None, :] * BNC

    hws, hbs = [], []
    for sp in params["split"]:
        (W0, b0), (W1, b1), (W2, b2) = sp
        w = jnp.zeros((320, 64), jnp.float32)
        w = w.at[0:192, :].set(W0)
        w = w.at[192:256, :].set(W1)
        w = w.at[256:320, 0:1].set(W2)
        b = jnp.zeros((3, 64), jnp.float32)
        b = b.at[0, :].set(b0)
        b = b.at[1, :].set(b1)
        b = b.at[2, 0].set(b2[0])
        hws.append(w)
        hbs.append(b)
    hw = jnp.stack(hws)
    hb = jnp.stack(hbs)

    return dict(w0c=w0c, b0c=b0c, layers=layers,
                d1=d1, db1=db1, d2=d2, db2=db2, d3=d3, db3=db3,
                hw=hw, hb=hb)


def kernel(x, edge_index, graph_ids, params):
    pp = _prep(params)
    idx_i = edge_index[:, 0].astype(jnp.int32)
    idx_j = edge_index[:, 1].astype(jnp.int32)
    zpad = jnp.zeros((EP - NE,), jnp.int32)
    iip = jnp.concatenate([idx_i, zpad])
    jjp = jnp.concatenate([idx_j, zpad])
    x_pad = jnp.pad(x, ((0, NP - NN), (0, 0)))

    t0 = _init_tables(x_pad, pp["w0c"], pp["b0c"])
    p = t0[:, 0:128]
    q = t0[:, 128:256]
    xcs = [None, t0[:, 256:384], t0[:, 384:512]]

    offs = jnp.searchsorted(idx_i, jnp.arange(NP + 24, dtype=jnp.int32),
                            side="left").astype(jnp.int32)
    goffs = jnp.searchsorted(graph_ids.astype(jnp.int32),
                             jnp.arange(NGRAPH + 24, dtype=jnp.int32),
                             side="left").astype(jnp.int32)
    cnt = (offs[1:NP + 1] - offs[:NP]).astype(jnp.float32)
    rcnt = (1.0 / jnp.maximum(cnt, 1.0))[:, None]
    gcnt = (goffs[1:NGRAPH + 1] - goffs[:NGRAPH]).astype(jnp.float32)
    rgcnt = (1.0 / jnp.maximum(gcnt, 1.0))[:, None]

    seg_reduce = _make_sc_reduce(NP, 320, 4, EP)
    pool_reduce = _make_sc_reduce(NGRAPH, 2, 1, NP, rw=64)

    h = None
    for l in range(3):
        lw = pp["layers"][l]
        pg, qg = _sc_gather(p, q, iip, jjp)
        m = _edge_mlp(pg, qg, lw["Wm2"], lw["bm2"])
        emb = seg_reduce(m, offs)
        if l < 2:
            nxt = (lw["A"], lw["Bw"], lw["pb"], xcs[l + 1])
            h, p, q = _node_update(emb, rcnt, lw["Wu1"], lw["bu1"],
                                   lw["Wu2"], lw["bu2"], nxt)
        else:
            h = _node_update(emb, rcnt, lw["Wu1"], lw["bu1"],
                             lw["Wu2"], lw["bu2"])

    pool_raw = pool_reduce(h, goffs)
    pool = pool_raw.reshape(32, 8, 512)[:, 0:2, :].reshape(NGRAPH, 512)
    return _decode(pool, rgcnt, pp["d1"], pp["db1"], pp["d2"],
                   pp["db2"], pp["d3"], pp["db3"], pp["hw"], pp["hb"])
